# baseline (speedup 1.0000x reference)
import functools

import jax
import jax.numpy as jnp
from jax import lax
from jax.experimental import pallas as pl
from jax.experimental.pallas import tpu as pltpu
from jax.experimental.pallas import tpu_sc as plsc

N = 10000
E = 320000
DIN = 128
H = 8
D = 16
ACC_W = 144
NC, NS = 2, 16
NW = NC * NS
EPW = E // NW
C = 40
NCHUNK = EPW // C
ROWS_PT = 632
NPAD = ROWS_PT * NS
RB = 1000


def _proj_body(x_ref, wq_ref, bq_ref, wk_ref, bk_ref, wv_ref, bv_ref,
               ws_ref, bs_ref, q_ref, k_ref, v_ref, sk_ref):
    xb = x_ref[...]
    q_ref[...] = (jnp.dot(xb, wq_ref[...], preferred_element_type=jnp.float32)
                  + bq_ref[...]) * jnp.float32(0.25)
    k_ref[...] = jnp.dot(xb, wk_ref[...], preferred_element_type=jnp.float32) + bk_ref[...]
    v_ref[...] = jnp.dot(xb, wv_ref[...], preferred_element_type=jnp.float32) + bv_ref[...]
    sk_ref[...] = jnp.dot(xb, ws_ref[...], preferred_element_type=jnp.float32) + bs_ref[...]


def _projections(x, Wq, bq, Wk, bk, Wv, bv, Ws, bs):
    row_spec = pl.BlockSpec((RB, DIN), lambda i: (i, 0))
    w_spec = pl.BlockSpec((DIN, DIN), lambda i: (0, 0))
    b_spec = pl.BlockSpec((1, DIN), lambda i: (0, 0))
    out = jax.ShapeDtypeStruct((N, DIN), jnp.float32)
    return pl.pallas_call(
        _proj_body,
        grid=(N // RB,),
        in_specs=[row_spec, w_spec, b_spec, w_spec, b_spec, w_spec, b_spec,
                  w_spec, b_spec],
        out_specs=[row_spec, row_spec, row_spec, row_spec],
        out_shape=[out, out, out, out],
    )(x, Wq, bq.reshape(1, DIN), Wk, bk.reshape(1, DIN),
      Wv, bv.reshape(1, DIN), Ws, bs.reshape(1, DIN))


def _edge_body(q_hbm, k_hbm, v_hbm, src_hbm, dst_hbm, zeros_hbm, out_hbm,
               acc_sh,
               sidx0, didx0, sidx1, didx1, sidx2, didx2, sidx3, didx3,
               qb0, kb0, vb0, qb1, kb1, vb1, msgbuf,
               isem0, isem1, isem2, isem3, gsem0, gsem1, ssem):
    c = lax.axis_index("c")
    s = lax.axis_index("s")
    row0 = pl.multiple_of(s * ROWS_PT, 8)
    pltpu.sync_copy(zeros_hbm.at[pl.ds(row0, ROWS_PT)],
                    acc_sh.at[pl.ds(row0, ROWS_PT)])
    plsc.subcore_barrier()

    lane = lax.iota(jnp.int32, 16)
    base = (c * NS + s) * EPW
    idxsets = [(sidx0, didx0, isem0), (sidx1, didx1, isem1),
               (sidx2, didx2, isem2), (sidx3, didx3, isem3)]
    gsets = [(qb0, kb0, vb0, gsem0), (qb1, kb1, vb1, gsem1)]

    def fetch_idx(i, m):
        sb, db, sem = idxsets[m]
        off = base + i * C
        pltpu.async_copy(src_hbm.at[pl.ds(off, C)], sb, sem)
        pltpu.async_copy(dst_hbm.at[pl.ds(off, C)], db, sem)

    def wait_idx(m):
        sb, db, sem = idxsets[m]
        pltpu.make_async_copy(src_hbm.at[pl.ds(0, C)], sb, sem).wait()
        pltpu.make_async_copy(dst_hbm.at[pl.ds(0, C)], db, sem).wait()

    def start_gathers(m, b):
        sb, db, _ = idxsets[m]
        qb, kb, vb, sem = gsets[b]
        pltpu.async_copy(q_hbm.at[sb], qb, sem)
        pltpu.async_copy(k_hbm.at[db], kb, sem)
        pltpu.async_copy(v_hbm.at[sb], vb, sem)

    def wait_gathers(b):
        qb, kb, vb, sem = gsets[b]
        pltpu.make_async_copy(q_hbm.at[sidx0], qb, sem).wait()
        pltpu.make_async_copy(k_hbm.at[didx0], kb, sem).wait()
        pltpu.make_async_copy(v_hbm.at[sidx0], vb, sem).wait()

    def scatter(m):
        db = idxsets[m][1]
        pltpu.async_copy(msgbuf, acc_sh.at[db], ssem, add=True)

    def wait_scatter():
        pltpu.make_async_copy(msgbuf, acc_sh.at[didx0], ssem).wait()

    def compute(qb, kb, vb):
        def edge_pair_body(i, carry2):
            e0 = i * 2
            es = (e0, e0 + 1)
            himask = jnp.int32(-65536)

            def unpack(ws):
                out = []
                for w in ws:
                    out.append(lax.bitcast_convert_type(
                        lax.shift_left(w, 16), jnp.float32))
                    out.append(lax.bitcast_convert_type(
                        lax.bitwise_and(w, himask), jnp.float32))
                return out

            ks = unpack([kb[e, pl.ds(j * D, D)]
                         for e in es for j in range(H // 2)])
            qs = unpack([qb[e, pl.ds(j * D, D)]
                         for e in es for j in range(H // 2)])
            ts = [qs[x * H + h] * ks[x * H + h]
                  for x, e in enumerate(es) for h in range(H)]
            for sh in (8, 4, 2, 1):
                ts = [t + t[lane ^ sh] for t in ts]
            exs = [jnp.exp(t) for t in ts]
            vs = unpack([vb[e, pl.ds(j * D, D)]
                         for e in es for j in range(H // 2)])
            for x, e in enumerate(es):
                for h in range(H):
                    msgbuf[e, pl.ds(h * D, D)] = (vs[x * H + h]
                                                  * exs[x * H + h])
            for x, e in enumerate(es):
                ex = exs[x * H:(x + 1) * H]
                a01 = jnp.where(lane == 0, ex[0], ex[1])
                a23 = jnp.where(lane == 2, ex[2], ex[3])
                a45 = jnp.where(lane == 4, ex[4], ex[5])
                a67 = jnp.where(lane == 6, ex[6], ex[7])
                b03 = jnp.where(lane < 2, a01, a23)
                b47 = jnp.where(lane < 6, a45, a67)
                c = jnp.where(lane < 4, b03, b47)
                msgbuf[e, pl.ds(DIN, 16)] = jnp.where(lane < H, c,
                                                      jnp.float32(0.0))
            return carry2

        lax.fori_loop(0, C // 2, edge_pair_body, 0)

    def phase(i, jmod4, do_gather_next, do_fetch2):
        b = jmod4 & 1
        wait_gathers(b)
        if do_gather_next:
            wait_idx((jmod4 + 1) % 4)
            start_gathers((jmod4 + 1) % 4, b ^ 1)
        wait_scatter()
        if do_fetch2:
            fetch_idx(i + 2, (jmod4 + 2) % 4)
        qb, kb, vb, _ = gsets[b]
        compute(qb, kb, vb)
        scatter(jmod4)

    def zrow_body(e, carry2):
        zv = jnp.zeros((16,), jnp.float32)
        for t16 in range(ACC_W // 16):
            msgbuf[e, pl.ds(t16 * 16, 16)] = zv
        return carry2

    lax.fori_loop(0, C, zrow_body, 0)
    fetch_idx(0, 0)
    wait_idx(0)
    start_gathers(0, 0)
    fetch_idx(1, 1)
    scatter(0)

    def loop_body(t, carry):
        i0 = t * 4
        for j in range(4):
            phase(i0 + j, j, True, True)
        return carry

    lax.fori_loop(0, (NCHUNK - 2) // 4, loop_body, 0)
    phase(NCHUNK - 2, (NCHUNK - 2) % 4, True, False)
    phase(NCHUNK - 1, (NCHUNK - 1) % 4, False, False)
    wait_scatter()

    plsc.subcore_barrier()
    pltpu.sync_copy(acc_sh.at[pl.ds(row0, ROWS_PT)],
                    out_hbm.at[c, pl.ds(row0, ROWS_PT)])


_edge_kernel = functools.partial(
    pl.kernel,
    out_type=jax.ShapeDtypeStruct((NC, NPAD, ACC_W), jnp.float32),
    mesh=plsc.VectorSubcoreMesh(core_axis_name="c", subcore_axis_name="s",
                                num_cores=NC, num_subcores=NS),
    compiler_params=pltpu.CompilerParams(use_tc_tiling_on_sc=False),
    scratch_types=[
        pltpu.VMEM_SHARED((NPAD, ACC_W), jnp.float32),
        pltpu.VMEM((C,), jnp.int32),
        pltpu.VMEM((C,), jnp.int32),
        pltpu.VMEM((C,), jnp.int32),
        pltpu.VMEM((C,), jnp.int32),
        pltpu.VMEM((C,), jnp.int32),
        pltpu.VMEM((C,), jnp.int32),
        pltpu.VMEM((C,), jnp.int32),
        pltpu.VMEM((C,), jnp.int32),
        pltpu.VMEM((C, DIN // 2), jnp.int32),
        pltpu.VMEM((C, DIN // 2), jnp.int32),
        pltpu.VMEM((C, DIN // 2), jnp.int32),
        pltpu.VMEM((C, DIN // 2), jnp.int32),
        pltpu.VMEM((C, DIN // 2), jnp.int32),
        pltpu.VMEM((C, DIN // 2), jnp.int32),
        pltpu.VMEM((C, ACC_W), jnp.float32),
        pltpu.SemaphoreType.DMA,
        pltpu.SemaphoreType.DMA,
        pltpu.SemaphoreType.DMA,
        pltpu.SemaphoreType.DMA,
        pltpu.SemaphoreType.DMA,
        pltpu.SemaphoreType.DMA,
        pltpu.SemaphoreType.DMA,
    ],
)(_edge_body)


def _epilogue_body(p_ref, sk_ref, w3_ref, bg_ref, gamma_ref, beta_ref, a_ref,
                   o_ref):
    agg = p_ref[0, :, 0:DIN] + p_ref[1, :, 0:DIN]
    den8 = p_ref[0, :, DIN:DIN + H] + p_ref[1, :, DIN:DIN + H]
    recip8 = 1.0 / (den8 + jnp.float32(1e-9))
    cols = [lax.broadcast_in_dim(recip8[:, h:h + 1], (RB, D), (0, 1))
            for h in range(H)]
    recip = jnp.concatenate(cols, axis=1)
    agg = agg * recip
    skip = sk_ref[...]
    w3 = w3_ref[...]
    wa = w3[0:1, :] + w3[2:3, :]
    wb = w3[1:2, :] - w3[2:3, :]
    z = jnp.sum(skip * wa + agg * wb, axis=1, keepdims=True) + bg_ref[0, 0]
    g = jax.nn.sigmoid(z)
    rst = g * skip + (1.0 - g) * agg
    mu = jnp.mean(rst, axis=1, keepdims=True)
    ctr = rst - mu
    var = jnp.mean(ctr * ctr, axis=1, keepdims=True)
    y = ctr * lax.rsqrt(var + jnp.float32(1e-5)) * gamma_ref[...] + beta_ref[...]
    o_ref[...] = jnp.where(y > 0, y, a_ref[0, 0] * y)


def _epilogue(parts, skip, Wg, bg, gamma, beta, prelu_a):
    row_spec = pl.BlockSpec((RB, DIN), lambda i: (i, 0))
    return pl.pallas_call(
        _epilogue_body,
        grid=(N // RB,),
        in_specs=[
            pl.BlockSpec((NC, RB, ACC_W), lambda i: (0, i, 0)),
            row_spec,
            pl.BlockSpec((3, DIN), lambda i: (0, 0)),
            pl.BlockSpec((1, 1), lambda i: (0, 0)),
            pl.BlockSpec((1, DIN), lambda i: (0, 0)),
            pl.BlockSpec((1, DIN), lambda i: (0, 0)),
            pl.BlockSpec((1, 1), lambda i: (0, 0)),
        ],
        out_specs=row_spec,
        out_shape=jax.ShapeDtypeStruct((N, DIN), jnp.float32),
    )(parts, skip, Wg.reshape(3, DIN), bg.reshape(1, 1),
      gamma.reshape(1, DIN), beta.reshape(1, DIN),
      jnp.asarray(prelu_a, jnp.float32).reshape(1, 1))


def _pack_bf16_pairs(a):
    z = a.reshape(N, H // 2, 2, D).transpose(0, 1, 3, 2)
    z = z.reshape(N, DIN).astype(jnp.bfloat16)
    return lax.bitcast_convert_type(z.reshape(N, DIN // 2, 2), jnp.int32)


def kernel(x, edge_index, Wq, bq, Wk, bk, Wv, bv, Ws, bs, Wg, bg, gamma, beta,
           prelu_a):
    q, k, v, skip = _projections(x, Wq, bq, Wk, bk, Wv, bv, Ws, bs)
    q = _pack_bf16_pairs(q)
    k = _pack_bf16_pairs(k)
    v = _pack_bf16_pairs(v)
    src = edge_index[0]
    dst = edge_index[1]
    zeros = jnp.zeros((NPAD, ACC_W), jnp.float32)
    parts = _edge_kernel(q, k, v, src, dst, zeros)
    return _epilogue(parts, skip, Wg, bg, gamma, beta, prelu_a)


# speedup vs baseline: 1.0602x; 1.0602x over previous
import functools

import numpy as np

import jax
import jax.numpy as jnp
from jax import lax
from jax.experimental import pallas as pl
from jax.experimental.pallas import tpu as pltpu
from jax.experimental.pallas import tpu_sc as plsc

N = 10000
E = 320000
DIN = 128
H = 8
D = 16
ACC_W = 144
NC, NS = 2, 16
NW = NC * NS
EPW = E // NW
C = 40
NCHUNK = EPW // C
ROWS_PT = 632
NPAD = ROWS_PT * NS
RB = 1000

_PAIR_PERM = np.zeros((DIN, DIN), np.float32)
for _j in range(H // 2):
    for _p in range(2):
        for _d in range(D):
            _PAIR_PERM[(2 * _j + _p) * D + _d, _p * 64 + _j * D + _d] = 1.0


def _proj_body(x_ref, wq_ref, bq_ref, wk_ref, bk_ref, wv_ref, bv_ref,
               ws_ref, bs_ref, q_ref, k_ref, v_ref, sk_ref):
    xb = x_ref[...]
    q_ref[...] = (jnp.dot(xb, wq_ref[...], preferred_element_type=jnp.float32)
                  + bq_ref[...]) * jnp.float32(0.25)
    def pack(mat):
        bits = lax.bitcast_convert_type(mat, jnp.int32)
        rnd = bits + jnp.int32(0x7FFF) + (
            lax.shift_right_logical(bits, 16) & jnp.int32(1))
        rnd = rnd & jnp.int32(-65536)
        lo = lax.shift_right_logical(rnd[:, 0:DIN // 2], 16)
        return rnd[:, DIN // 2:DIN] | lo

    kmat = jnp.dot(xb, wk_ref[...], preferred_element_type=jnp.float32) + bk_ref[...]
    k_ref[...] = pack(kmat)
    vmat = jnp.dot(xb, wv_ref[...], preferred_element_type=jnp.float32) + bv_ref[...]
    v_ref[...] = pack(vmat)
    sk_ref[...] = jnp.dot(xb, ws_ref[...], preferred_element_type=jnp.float32) + bs_ref[...]


def _projections(x, Wq, bq, Wk, bk, Wv, bv, Ws, bs):
    row_spec = pl.BlockSpec((RB, DIN), lambda i: (i, 0))
    w_spec = pl.BlockSpec((DIN, DIN), lambda i: (0, 0))
    b_spec = pl.BlockSpec((1, DIN), lambda i: (0, 0))
    pk_spec = pl.BlockSpec((RB, DIN // 2), lambda i: (i, 0))
    out = jax.ShapeDtypeStruct((N, DIN), jnp.float32)
    outp = jax.ShapeDtypeStruct((N, DIN // 2), jnp.int32)
    return pl.pallas_call(
        _proj_body,
        grid=(N // RB,),
        in_specs=[row_spec, w_spec, b_spec, w_spec, b_spec, w_spec, b_spec,
                  w_spec, b_spec],
        out_specs=[row_spec, pk_spec, pk_spec, row_spec],
        out_shape=[out, outp, outp, out],
    )(x, Wq, bq.reshape(1, DIN),
      Wk @ _PAIR_PERM, (bk @ _PAIR_PERM).reshape(1, DIN),
      Wv @ _PAIR_PERM, (bv @ _PAIR_PERM).reshape(1, DIN),
      Ws, bs.reshape(1, DIN))


def _edge_body(q_hbm, k_hbm, v_hbm, src_hbm, dst_hbm, zeros_hbm, out_hbm,
               acc_sh,
               sidx0, didx0, sidx1, didx1, sidx2, didx2, sidx3, didx3,
               qb0, kb0, vb0, qb1, kb1, vb1, msgbuf,
               isem0, isem1, isem2, isem3, gsem0, gsem1, ssem):
    c = lax.axis_index("c")
    s = lax.axis_index("s")
    row0 = pl.multiple_of(s * ROWS_PT, 8)
    pltpu.sync_copy(zeros_hbm.at[pl.ds(row0, ROWS_PT)],
                    acc_sh.at[pl.ds(row0, ROWS_PT)])
    plsc.subcore_barrier()

    lane = lax.iota(jnp.int32, 16)
    base = (c * NS + s) * EPW
    idxsets = [(sidx0, didx0, isem0), (sidx1, didx1, isem1),
               (sidx2, didx2, isem2), (sidx3, didx3, isem3)]
    gsets = [(qb0, kb0, vb0, gsem0), (qb1, kb1, vb1, gsem1)]

    def fetch_idx(i, m):
        sb, db, sem = idxsets[m]
        off = base + i * C
        pltpu.async_copy(src_hbm.at[pl.ds(off, C)], sb, sem)
        pltpu.async_copy(dst_hbm.at[pl.ds(off, C)], db, sem)

    def wait_idx(m):
        sb, db, sem = idxsets[m]
        pltpu.make_async_copy(src_hbm.at[pl.ds(0, C)], sb, sem).wait()
        pltpu.make_async_copy(dst_hbm.at[pl.ds(0, C)], db, sem).wait()

    def start_gathers(m, b):
        sb, db, _ = idxsets[m]
        qb, kb, vb, sem = gsets[b]
        pltpu.async_copy(q_hbm.at[sb], qb, sem)
        pltpu.async_copy(k_hbm.at[db], kb, sem)
        pltpu.async_copy(v_hbm.at[sb], vb, sem)

    def wait_gathers(b):
        qb, kb, vb, sem = gsets[b]
        pltpu.make_async_copy(q_hbm.at[sidx0], qb, sem).wait()
        pltpu.make_async_copy(k_hbm.at[didx0], kb, sem).wait()
        pltpu.make_async_copy(v_hbm.at[sidx0], vb, sem).wait()

    def scatter(m):
        db = idxsets[m][1]
        pltpu.async_copy(msgbuf, acc_sh.at[db], ssem, add=True)

    def wait_scatter():
        pltpu.make_async_copy(msgbuf, acc_sh.at[didx0], ssem).wait()

    def compute(qb, kb, vb):
        def edge_pair_body(i, carry2):
            e0 = i * 2
            es = (e0, e0 + 1)
            himask = jnp.int32(-65536)

            def unpack(ws):
                out = []
                for w in ws:
                    out.append(lax.bitcast_convert_type(
                        lax.shift_left(w, 16), jnp.float32))
                    out.append(lax.bitcast_convert_type(
                        lax.bitwise_and(w, himask), jnp.float32))
                return out

            ks = unpack([kb[e, pl.ds(j * D, D)]
                         for e in es for j in range(H // 2)])
            ts = [qb[e, pl.ds(h * D, D)] * ks[x * H + h]
                  for x, e in enumerate(es) for h in range(H)]
            for sh in (8, 4, 2, 1):
                ts = [t + t[lane ^ sh] for t in ts]
            exs = [jnp.exp(t) for t in ts]
            vs = unpack([vb[e, pl.ds(j * D, D)]
                         for e in es for j in range(H // 2)])
            for x, e in enumerate(es):
                for h in range(H):
                    msgbuf[e, pl.ds(h * D, D)] = (vs[x * H + h]
                                                  * exs[x * H + h])
            for x, e in enumerate(es):
                ex = exs[x * H:(x + 1) * H]
                a01 = jnp.where(lane == 0, ex[0], ex[1])
                a23 = jnp.where(lane == 2, ex[2], ex[3])
                a45 = jnp.where(lane == 4, ex[4], ex[5])
                a67 = jnp.where(lane == 6, ex[6], ex[7])
                b03 = jnp.where(lane < 2, a01, a23)
                b47 = jnp.where(lane < 6, a45, a67)
                c = jnp.where(lane < 4, b03, b47)
                msgbuf[e, pl.ds(DIN, 16)] = jnp.where(lane < H, c,
                                                      jnp.float32(0.0))
            return carry2

        lax.fori_loop(0, C // 2, edge_pair_body, 0)

    def phase(i, jmod4, do_gather_next, do_fetch2):
        b = jmod4 & 1
        wait_gathers(b)
        if do_gather_next:
            wait_idx((jmod4 + 1) % 4)
            start_gathers((jmod4 + 1) % 4, b ^ 1)
        wait_scatter()
        if do_fetch2:
            fetch_idx(i + 2, (jmod4 + 2) % 4)
        qb, kb, vb, _ = gsets[b]
        compute(qb, kb, vb)
        scatter(jmod4)

    def zrow_body(e, carry2):
        zv = jnp.zeros((16,), jnp.float32)
        for t16 in range(ACC_W // 16):
            msgbuf[e, pl.ds(t16 * 16, 16)] = zv
        return carry2

    lax.fori_loop(0, C, zrow_body, 0)
    fetch_idx(0, 0)
    wait_idx(0)
    start_gathers(0, 0)
    fetch_idx(1, 1)
    scatter(0)

    def loop_body(t, carry):
        i0 = t * 4
        for j in range(4):
            phase(i0 + j, j, True, True)
        return carry

    lax.fori_loop(0, (NCHUNK - 2) // 4, loop_body, 0)
    phase(NCHUNK - 2, (NCHUNK - 2) % 4, True, False)
    phase(NCHUNK - 1, (NCHUNK - 1) % 4, False, False)
    wait_scatter()

    plsc.subcore_barrier()
    pltpu.sync_copy(acc_sh.at[pl.ds(row0, ROWS_PT)],
                    out_hbm.at[c, pl.ds(row0, ROWS_PT)])


_edge_kernel = functools.partial(
    pl.kernel,
    out_type=jax.ShapeDtypeStruct((NC, NPAD, ACC_W), jnp.float32),
    mesh=plsc.VectorSubcoreMesh(core_axis_name="c", subcore_axis_name="s",
                                num_cores=NC, num_subcores=NS),
    compiler_params=pltpu.CompilerParams(use_tc_tiling_on_sc=False),
    scratch_types=[
        pltpu.VMEM_SHARED((NPAD, ACC_W), jnp.float32),
        pltpu.VMEM((C,), jnp.int32),
        pltpu.VMEM((C,), jnp.int32),
        pltpu.VMEM((C,), jnp.int32),
        pltpu.VMEM((C,), jnp.int32),
        pltpu.VMEM((C,), jnp.int32),
        pltpu.VMEM((C,), jnp.int32),
        pltpu.VMEM((C,), jnp.int32),
        pltpu.VMEM((C,), jnp.int32),
        pltpu.VMEM((C, DIN), jnp.float32),
        pltpu.VMEM((C, DIN // 2), jnp.int32),
        pltpu.VMEM((C, DIN // 2), jnp.int32),
        pltpu.VMEM((C, DIN), jnp.float32),
        pltpu.VMEM((C, DIN // 2), jnp.int32),
        pltpu.VMEM((C, DIN // 2), jnp.int32),
        pltpu.VMEM((C, ACC_W), jnp.float32),
        pltpu.SemaphoreType.DMA,
        pltpu.SemaphoreType.DMA,
        pltpu.SemaphoreType.DMA,
        pltpu.SemaphoreType.DMA,
        pltpu.SemaphoreType.DMA,
        pltpu.SemaphoreType.DMA,
        pltpu.SemaphoreType.DMA,
    ],
)(_edge_body)


def _epilogue_body(p_ref, sk_ref, w3_ref, bg_ref, gamma_ref, beta_ref, a_ref,
                   o_ref):
    agg = p_ref[0, :, 0:DIN] + p_ref[1, :, 0:DIN]
    den8 = p_ref[0, :, DIN:DIN + H] + p_ref[1, :, DIN:DIN + H]
    recip8 = 1.0 / (den8 + jnp.float32(1e-9))
    cols = [lax.broadcast_in_dim(recip8[:, h:h + 1], (RB, D), (0, 1))
            for h in range(H)]
    recip = jnp.concatenate(cols, axis=1)
    agg = agg * recip
    skip = sk_ref[...]
    w3 = w3_ref[...]
    wa = w3[0:1, :] + w3[2:3, :]
    wb = w3[1:2, :] - w3[2:3, :]
    z = jnp.sum(skip * wa + agg * wb, axis=1, keepdims=True) + bg_ref[0, 0]
    g = jax.nn.sigmoid(z)
    rst = g * skip + (1.0 - g) * agg
    mu = jnp.mean(rst, axis=1, keepdims=True)
    ctr = rst - mu
    var = jnp.mean(ctr * ctr, axis=1, keepdims=True)
    y = ctr * lax.rsqrt(var + jnp.float32(1e-5)) * gamma_ref[...] + beta_ref[...]
    o_ref[...] = jnp.where(y > 0, y, a_ref[0, 0] * y)


def _epilogue(parts, skip, Wg, bg, gamma, beta, prelu_a):
    row_spec = pl.BlockSpec((RB, DIN), lambda i: (i, 0))
    return pl.pallas_call(
        _epilogue_body,
        grid=(N // RB,),
        in_specs=[
            pl.BlockSpec((NC, RB, ACC_W), lambda i: (0, i, 0)),
            row_spec,
            pl.BlockSpec((3, DIN), lambda i: (0, 0)),
            pl.BlockSpec((1, 1), lambda i: (0, 0)),
            pl.BlockSpec((1, DIN), lambda i: (0, 0)),
            pl.BlockSpec((1, DIN), lambda i: (0, 0)),
            pl.BlockSpec((1, 1), lambda i: (0, 0)),
        ],
        out_specs=row_spec,
        out_shape=jax.ShapeDtypeStruct((N, DIN), jnp.float32),
    )(parts, skip, Wg.reshape(3, DIN), bg.reshape(1, 1),
      gamma.reshape(1, DIN), beta.reshape(1, DIN),
      jnp.asarray(prelu_a, jnp.float32).reshape(1, 1))


def kernel(x, edge_index, Wq, bq, Wk, bk, Wv, bv, Ws, bs, Wg, bg, gamma, beta,
           prelu_a):
    q, k, v, skip = _projections(x, Wq, bq, Wk, bk, Wv, bv, Ws, bs)

    src = edge_index[0]
    dst = edge_index[1]
    zeros = jnp.zeros((NPAD, ACC_W), jnp.float32)
    parts = _edge_kernel(q, k, v, src, dst, zeros)
    return _epilogue(parts, skip, Wg, bg, gamma, beta, prelu_a)
